# Initial kernel scaffold; baseline (speedup 1.0000x reference)
import functools

import jax
import jax.numpy as jnp
from jax import lax
from jax.experimental import pallas as pl
from jax.experimental.pallas import tpu as pltpu
from jax.experimental.pallas import tpu_sc as plsc

N = 10000
NE = 10000
E = 320000
D = 128

NC = 2
NS = 16
NW = NC * NS
T = E // NW
K = 80
NCHUNK = T // K
S_PAD = 10240
ROWS_PER_TILE = S_PAD // NS


def _agg_body(table, gidx, sidx, zeros, out, gidx_v, sidx_v, rows_v, acc_sh, sem):
    c = lax.axis_index("c")
    s = lax.axis_index("s")
    wid = c * NS + s
    pltpu.sync_copy(gidx.at[wid], gidx_v)
    pltpu.sync_copy(sidx.at[wid], sidx_v)
    pltpu.sync_copy(zeros, acc_sh.at[pl.ds(s * ROWS_PER_TILE, ROWS_PER_TILE)])
    plsc.subcore_barrier()

    def step(j, carry):
        pltpu.async_copy(table.at[gidx_v.at[j]], rows_v, sem).wait()
        pltpu.sync_copy(rows_v, acc_sh.at[sidx_v.at[j]], add=True)
        return carry

    lax.fori_loop(0, NCHUNK, step, 0)
    plsc.subcore_barrier()
    sl = pl.ds(s * ROWS_PER_TILE, ROWS_PER_TILE)
    pltpu.sync_copy(acc_sh.at[sl], out.at[c, sl])


def _make_agg():
    mesh = plsc.VectorSubcoreMesh(core_axis_name="c", subcore_axis_name="s")
    return pl.kernel(
        _agg_body,
        out_type=jax.ShapeDtypeStruct((NC, S_PAD, D), jnp.float32),
        mesh=mesh,
        scratch_types=[
            pltpu.VMEM((NCHUNK, K), jnp.int32),
            pltpu.VMEM((NCHUNK, K), jnp.int32),
            pltpu.VMEM((K, D), jnp.float32),
            pltpu.VMEM_SHARED((S_PAD, D), jnp.float32),
            pltpu.SemaphoreType.DMA,
        ],
    )


def _cnt_body(eidx, nidx, ones, zeros, eout, nout,
              eidx_v, nidx_v, ones_v, ecnt_sh, ncnt_sh):
    c = lax.axis_index("c")
    s = lax.axis_index("s")
    wid = c * NS + s
    pltpu.sync_copy(eidx.at[wid], eidx_v)
    pltpu.sync_copy(nidx.at[wid], nidx_v)
    pltpu.sync_copy(ones, ones_v)
    sl = pl.ds(s * ROWS_PER_TILE, ROWS_PER_TILE)
    pltpu.sync_copy(zeros, ecnt_sh.at[sl])
    pltpu.sync_copy(zeros, ncnt_sh.at[sl])
    plsc.subcore_barrier()

    def step(j, carry):
        pltpu.sync_copy(ones_v, ecnt_sh.at[eidx_v.at[j]], add=True)
        pltpu.sync_copy(ones_v, ncnt_sh.at[nidx_v.at[j]], add=True)
        return carry

    lax.fori_loop(0, NCHUNK, step, 0)
    plsc.subcore_barrier()
    pltpu.sync_copy(ecnt_sh.at[sl], eout.at[c, sl])
    pltpu.sync_copy(ncnt_sh.at[sl], nout.at[c, sl])


def _make_cnt():
    mesh = plsc.VectorSubcoreMesh(core_axis_name="c", subcore_axis_name="s")
    return pl.kernel(
        _cnt_body,
        out_type=(
            jax.ShapeDtypeStruct((NC, S_PAD, 16), jnp.float32),
            jax.ShapeDtypeStruct((NC, S_PAD, 16), jnp.float32),
        ),
        mesh=mesh,
        scratch_types=[
            pltpu.VMEM((NCHUNK, K), jnp.int32),
            pltpu.VMEM((NCHUNK, K), jnp.int32),
            pltpu.VMEM((K, 16), jnp.float32),
            pltpu.VMEM_SHARED((S_PAD, 16), jnp.float32),
            pltpu.VMEM_SHARED((S_PAD, 16), jnp.float32),
        ],
    )


def _combine_body(relu, p_ref, cnt_ref, w_ref, b_ref, o_ref):
    ssum = p_ref[0] + p_ref[1]
    cnt = cnt_ref[0, :, 0:1] + cnt_ref[1, :, 0:1]
    mean = ssum / jnp.maximum(cnt, 1.0)
    y = jnp.dot(mean, w_ref[...], preferred_element_type=jnp.float32)
    y = y[:NE] + b_ref[...][None, :]
    if relu:
        y = jnp.maximum(y, 0.0)
    o_ref[...] = y


def _combine(partials, cnts, w, b, relu):
    body = functools.partial(_combine_body, relu)
    return pl.pallas_call(
        body,
        out_shape=jax.ShapeDtypeStruct((NE, D), jnp.float32),
    )(partials, cnts, w, b)


def kernel(x, ei, W1_e, b1_e, W1_n, b1_n, W2_e, b2_e, W2_n, b2_n):
    nid3 = ei[0].reshape(NW, NCHUNK, K)
    eid3 = ei[1].reshape(NW, NCHUNK, K)
    zeros_d = jnp.zeros((ROWS_PER_TILE, D), jnp.float32)
    zeros_c = jnp.zeros((ROWS_PER_TILE, 16), jnp.float32)
    ones_c = jnp.ones((K, 16), jnp.float32)

    agg = _make_agg()
    ecnt_p, ncnt_p = _make_cnt()(eid3, nid3, ones_c, zeros_c)

    h = x
    for (We, be, Wn, bn) in ((W1_e, b1_e, W1_n, b1_n), (W2_e, b2_e, W2_n, b2_n)):
        ep = agg(h, nid3, eid3, zeros_d)
        ef = _combine(ep, ecnt_p, We, be, relu=False)
        np_ = agg(ef, eid3, nid3, zeros_d)
        h = _combine(np_, ncnt_p, Wn, bn, relu=True)
    return h


# speedup vs baseline: 4.5420x; 4.5420x over previous
import functools

import jax
import jax.numpy as jnp
from jax import lax
from jax.experimental import pallas as pl
from jax.experimental.pallas import tpu as pltpu
from jax.experimental.pallas import tpu_sc as plsc

N = 10000
NE = 10000
E = 320000
D = 128

NC = 2
NS = 16
NW = NC * NS
T = E // NW
K = 80
NCHUNK = T // K
S_PAD = 10240
ROWS_PER_TILE = S_PAD // NS
BR = 16


def _agg_body(table, gidx, sidx, zeros, out, gidx_v, sidx_v, rows_v, bounce_v,
              acc_sh, sem):
    c = lax.axis_index("c")
    s = lax.axis_index("s")
    wid = c * NS + s
    pltpu.sync_copy(gidx.at[wid], gidx_v)
    pltpu.sync_copy(sidx.at[wid], sidx_v)
    pltpu.sync_copy(zeros, bounce_v)
    for r in range(ROWS_PER_TILE // BR):
        pltpu.sync_copy(bounce_v,
                        acc_sh.at[pl.ds(s * ROWS_PER_TILE + r * BR, BR)])
    plsc.subcore_barrier()

    def step(j, carry):
        pltpu.async_copy(table.at[gidx_v.at[j]], rows_v, sem).wait()
        pltpu.sync_copy(rows_v, acc_sh.at[sidx_v.at[j]], add=True)
        return carry

    lax.fori_loop(0, NCHUNK, step, 0)
    plsc.subcore_barrier()
    for r in range(ROWS_PER_TILE // BR):
        sl = pl.ds(s * ROWS_PER_TILE + r * BR, BR)
        pltpu.sync_copy(acc_sh.at[sl], bounce_v)
        pltpu.sync_copy(bounce_v, out.at[c, sl])


def _make_agg():
    mesh = plsc.VectorSubcoreMesh(core_axis_name="c", subcore_axis_name="s")
    return pl.kernel(
        _agg_body,
        out_type=jax.ShapeDtypeStruct((NC, S_PAD, D), jnp.float32),
        mesh=mesh,
        scratch_types=[
            pltpu.VMEM((NCHUNK, K), jnp.int32),
            pltpu.VMEM((NCHUNK, K), jnp.int32),
            pltpu.VMEM((K, D), jnp.float32),
            pltpu.VMEM((BR, D), jnp.float32),
            pltpu.VMEM_SHARED((S_PAD, D), jnp.float32),
            pltpu.SemaphoreType.DMA,
        ],
    )


def _combine_body(relu, p_ref, cnt_ref, w_ref, b_ref, o_ref):
    ssum = p_ref[0] + p_ref[1]
    cnt = cnt_ref[0] + cnt_ref[1]
    mean = ssum / jnp.maximum(cnt, 1.0)
    y = jnp.dot(mean, w_ref[...], preferred_element_type=jnp.float32)
    y = y[:NE] + b_ref[...][None, :]
    if relu:
        y = jnp.maximum(y, 0.0)
    o_ref[...] = y


def _combine(partials, cnts, w, b, relu):
    body = functools.partial(_combine_body, relu)
    return pl.pallas_call(
        body,
        out_shape=jax.ShapeDtypeStruct((NE, D), jnp.float32),
    )(partials, cnts, w, b)


def kernel(x, ei, W1_e, b1_e, W1_n, b1_n, W2_e, b2_e, W2_n, b2_n):
    nid3 = ei[0].reshape(NW, NCHUNK, K)
    eid3 = ei[1].reshape(NW, NCHUNK, K)
    zeros_b = jnp.zeros((BR, D), jnp.float32)
    ones_t = jnp.ones((N, D), jnp.float32)

    agg = _make_agg()
    cnt_e = agg(ones_t, nid3, eid3, zeros_b)
    cnt_n = agg(ones_t, eid3, nid3, zeros_b)

    h = x
    for (We, be, Wn, bn) in ((W1_e, b1_e, W1_n, b1_n), (W2_e, b2_e, W2_n, b2_n)):
        ep = agg(h, nid3, eid3, zeros_b)
        ef = _combine(ep, cnt_e, We, be, relu=False)
        np_ = agg(ef, eid3, nid3, zeros_b)
        h = _combine(np_, cnt_n, Wn, bn, relu=True)
    return h
